# Initial kernel scaffold; baseline (speedup 1.0000x reference)
import jax
import jax.numpy as jnp
from jax.experimental import pallas as pl


def kernel(x_rows, x_cols, x_vals, emb):
    raise NotImplementedError("write your pallas kernel here")


# speedup vs baseline: 20.0810x; 20.0810x over previous
import functools

import jax
import jax.numpy as jnp
from jax import lax
from jax.experimental import pallas as pl
from jax.experimental.pallas import tpu as pltpu
from jax.experimental.pallas import tpu_sc as plsc

_N_CELLS = 16384
_N_GENES = 2000
_NUM_HID = 128
_KP = 2048
_NW = 32
_R_BUF = 32
_CHUNKS = _N_CELLS // _R_BUF
_CPW = _CHUNKS // _NW
_NB = 2048
_LN2 = 0.6931471805599453


def _log1p_sc(v):
    u = v + 1.0
    m = u
    ef = jnp.zeros_like(v)
    for shift, p2 in ((8.0, 256.0), (4.0, 16.0), (2.0, 4.0), (1.0, 2.0)):
        big = m >= p2
        m = jnp.where(big, m * (1.0 / p2), m)
        ef = ef + jnp.where(big, shift, 0.0)
    big = m >= 1.4142135
    m = jnp.where(big, m * 0.5, m)
    ef = ef + jnp.where(big, 1.0, 0.0)
    s = (m - 1.0) / (m + 1.0)
    z = s * s
    p = 2.0 * s * (1.0 + z * (1.0 / 3.0 + z * (0.2 + z * (1.0 / 7.0))))
    return ef * _LN2 + p


def _densify_body(rows_hbm, cols_hbm, vals_hbm, starts_hbm, ends_hbm,
                  a_hbm, abuf, rbuf, cbuf, vbuf, sbuf, ebuf):
    w = lax.axis_index("s") * 2 + lax.axis_index("c")
    wb = pl.multiple_of(w * _CPW, 8)
    pltpu.sync_copy(starts_hbm.at[pl.ds(wb, _CPW)], sbuf)
    pltpu.sync_copy(ends_hbm.at[pl.ds(wb, _CPW)], ebuf)
    sv = sbuf[...]
    ev = ebuf[...]

    def chunk_body(g_local):
        g = w * _CPW + g_local
        r_lo = g * _R_BUF

        def zero_body(i, _):
            abuf[pl.ds(i * 16, 16)] = jnp.zeros((16,), jnp.float32)
            return 0

        lax.fori_loop(0, _R_BUF * _KP // 16, zero_body, 0)

        n0 = sv[g_local]
        n1 = ev[g_local]
        trips = (n1 - n0 + (_NB - 1)) // _NB

        def nnz_body(t, _):
            na = pl.multiple_of(n0 + t * _NB, 8)
            pltpu.sync_copy(rows_hbm.at[pl.ds(na, _NB)], rbuf)
            pltpu.sync_copy(cols_hbm.at[pl.ds(na, _NB)], cbuf)
            pltpu.sync_copy(vals_hbm.at[pl.ds(na, _NB)], vbuf)

            def grp(j, _):
                r = rbuf[pl.ds(j * 16, 16)]
                c = cbuf[pl.ds(j * 16, 16)]
                v = vbuf[pl.ds(j * 16, 16)]
                msk = (r >= r_lo) & (r < r_lo + _R_BUF)
                lv = jnp.where(msk, _log1p_sc(v), 0.0)
                idx = ((r - r_lo) << 11) + c
                idx = jnp.where(msk, idx, 0)
                plsc.addupdate_scatter(abuf, [idx], lv, mask=msk)
                return 0

            lax.fori_loop(0, _NB // 16, grp, 0)
            return 0

        lax.fori_loop(0, trips, nnz_body, 0)
        ga = pl.multiple_of(g * _R_BUF * _KP, 8)
        pltpu.sync_copy(abuf, a_hbm.at[pl.ds(ga, _R_BUF * _KP)])

    for g_local in range(_CPW):
        chunk_body(g_local)


def _densify(rows_p, cols_p, vals_p, starts, ends):
    mesh = plsc.VectorSubcoreMesh(core_axis_name="c", subcore_axis_name="s")
    return pl.kernel(
        _densify_body,
        out_type=jax.ShapeDtypeStruct((_N_CELLS * _KP,), jnp.float32),
        mesh=mesh,
        compiler_params=pltpu.CompilerParams(needs_layout_passes=False),
        scratch_types=[
            pltpu.VMEM((_R_BUF * _KP,), jnp.float32),
            pltpu.VMEM((_NB,), jnp.int32),
            pltpu.VMEM((_NB,), jnp.int32),
            pltpu.VMEM((_NB,), jnp.float32),
            pltpu.VMEM((_CPW,), jnp.int32),
            pltpu.VMEM((_CPW,), jnp.int32),
        ],
    )(rows_p, cols_p, vals_p, starts, ends)


def _matmul_body(a_ref, b_ref, o_ref):
    o_ref[...] = jnp.dot(a_ref[...], b_ref[...],
                         preferred_element_type=jnp.float32)


_BM = 256


def _matmul(a, emb_pad):
    return pl.pallas_call(
        _matmul_body,
        grid=(_N_CELLS // _BM,),
        in_specs=[
            pl.BlockSpec((_BM, _KP), lambda i: (i, 0)),
            pl.BlockSpec((_KP, _NUM_HID), lambda i: (0, 0)),
        ],
        out_specs=pl.BlockSpec((_BM, _NUM_HID), lambda i: (i, 0)),
        out_shape=jax.ShapeDtypeStruct((_N_CELLS, _NUM_HID), jnp.float32),
    )(a, emb_pad)


def kernel(x_rows, x_cols, x_vals, emb):
    edges = jnp.arange(0, _N_CELLS + 1, _R_BUF, dtype=jnp.int32)
    bounds = jnp.searchsorted(x_rows, edges, side="left").astype(jnp.int32)
    starts = bounds[:-1] & jnp.int32(~7)
    ends = bounds[1:]
    rows_p = jnp.concatenate([x_rows, jnp.full((_NB,), _N_CELLS, jnp.int32)])
    cols_p = jnp.concatenate([x_cols, jnp.zeros((_NB,), jnp.int32)])
    vals_p = jnp.concatenate([x_vals, jnp.zeros((_NB,), jnp.float32)])

    a_flat = _densify(rows_p, cols_p, vals_p, starts, ends)
    a = a_flat.reshape(_N_CELLS, _KP)
    emb_pad = jnp.pad(emb, ((0, _KP - _N_GENES), (0, 0)))
    return _matmul(a, emb_pad)


# speedup vs baseline: 33.5173x; 1.6691x over previous
import functools

import jax
import jax.numpy as jnp
from jax import lax
from jax.experimental import pallas as pl
from jax.experimental.pallas import tpu as pltpu
from jax.experimental.pallas import tpu_sc as plsc

_N_CELLS = 16384
_N_GENES = 2000
_NUM_HID = 128
_KP = 2048
_NW = 32
_R_BUF = 32
_CHUNKS = _N_CELLS // _R_BUF
_CPW = _CHUNKS // _NW
_NB = 2048
_LN2 = 0.6931471805599453


def _log1p_body(v_ref, o_ref):
    o_ref[...] = jnp.log1p(v_ref[...])


def _log1p_tc(v):
    v2 = v.reshape(25600, 128)
    out = pl.pallas_call(
        _log1p_body,
        grid=(16,),
        in_specs=[pl.BlockSpec((1600, 128), lambda i: (i, 0))],
        out_specs=pl.BlockSpec((1600, 128), lambda i: (i, 0)),
        out_shape=jax.ShapeDtypeStruct((25600, 128), jnp.float32),
    )(v2)
    return out.reshape(-1)


def _densify_body(rows_hbm, cols_hbm, vals_hbm, starts_hbm, ends_hbm,
                  a_hbm, abuf, rbuf, cbuf, vbuf, sbuf, ebuf,
                  sem_r, sem_c, sem_v):
    w = lax.axis_index("s") * 2 + lax.axis_index("c")
    wb = pl.multiple_of(w * _CPW, 8)
    pltpu.sync_copy(starts_hbm.at[pl.ds(wb, _CPW)], sbuf)
    pltpu.sync_copy(ends_hbm.at[pl.ds(wb, _CPW)], ebuf)
    sv = sbuf[...]
    ev = ebuf[...]
    zeros16 = jnp.zeros((16,), jnp.float32)

    def chunk_body(g_local):
        g = w * _CPW + g_local
        r_lo = g * _R_BUF

        def zero_body(i, _):
            for u in range(16):
                abuf[pl.ds(i * 256 + u * 16, 16)] = zeros16
            return 0

        lax.fori_loop(0, _R_BUF * _KP // 256, zero_body, 0)

        n0 = sv[g_local]
        n1 = ev[g_local]
        trips = (n1 - n0 + (_NB - 1)) // _NB

        def nnz_body(t, _):
            na = pl.multiple_of(n0 + t * _NB, 8)
            cp_r = pltpu.async_copy(rows_hbm.at[pl.ds(na, _NB)], rbuf, sem_r)
            cp_c = pltpu.async_copy(cols_hbm.at[pl.ds(na, _NB)], cbuf, sem_c)
            cp_v = pltpu.async_copy(vals_hbm.at[pl.ds(na, _NB)], vbuf, sem_v)
            cp_r.wait()
            cp_c.wait()
            cp_v.wait()

            def grp(j, _):
                for u in range(2):
                    r = rbuf[pl.ds(j * 32 + u * 16, 16)]
                    c = cbuf[pl.ds(j * 32 + u * 16, 16)]
                    lv = vbuf[pl.ds(j * 32 + u * 16, 16)]
                    msk = (r >= r_lo) & (r < r_lo + _R_BUF)
                    idx = ((r - r_lo) << 11) + c
                    idx = jnp.where(msk, idx, 0)
                    plsc.addupdate_scatter(abuf, [idx], lv, mask=msk)
                return 0

            lax.fori_loop(0, _NB // 32, grp, 0)
            return 0

        lax.fori_loop(0, trips, nnz_body, 0)
        ga = pl.multiple_of(g * _R_BUF * _KP, 8)
        pltpu.sync_copy(abuf, a_hbm.at[pl.ds(ga, _R_BUF * _KP)])

    for g_local in range(_CPW):
        chunk_body(g_local)


def _densify(rows_p, cols_p, vals_p, starts, ends):
    mesh = plsc.VectorSubcoreMesh(core_axis_name="c", subcore_axis_name="s")
    return pl.kernel(
        _densify_body,
        out_type=jax.ShapeDtypeStruct((_N_CELLS * _KP,), jnp.float32),
        mesh=mesh,
        compiler_params=pltpu.CompilerParams(needs_layout_passes=False),
        scratch_types=[
            pltpu.VMEM((_R_BUF * _KP,), jnp.float32),
            pltpu.VMEM((_NB,), jnp.int32),
            pltpu.VMEM((_NB,), jnp.int32),
            pltpu.VMEM((_NB,), jnp.float32),
            pltpu.VMEM((_CPW,), jnp.int32),
            pltpu.VMEM((_CPW,), jnp.int32),
            pltpu.SemaphoreType.DMA,
            pltpu.SemaphoreType.DMA,
            pltpu.SemaphoreType.DMA,
        ],
    )(rows_p, cols_p, vals_p, starts, ends)


def _matmul_body(a_ref, b_ref, o_ref):
    o_ref[...] = jnp.dot(a_ref[...], b_ref[...],
                         preferred_element_type=jnp.float32)


_BM = 256


def _matmul(a, emb_pad):
    return pl.pallas_call(
        _matmul_body,
        grid=(_N_CELLS // _BM,),
        in_specs=[
            pl.BlockSpec((_BM, _KP), lambda i: (i, 0)),
            pl.BlockSpec((_KP, _NUM_HID), lambda i: (0, 0)),
        ],
        out_specs=pl.BlockSpec((_BM, _NUM_HID), lambda i: (i, 0)),
        out_shape=jax.ShapeDtypeStruct((_N_CELLS, _NUM_HID), jnp.float32),
    )(a, emb_pad)


def kernel(x_rows, x_cols, x_vals, emb):
    edges = jnp.arange(0, _N_CELLS + 1, _R_BUF, dtype=jnp.int32)
    bounds = jnp.searchsorted(x_rows, edges, side="left").astype(jnp.int32)
    starts = bounds[:-1] & jnp.int32(~7)
    ends = bounds[1:]
    rows_p = jnp.concatenate([x_rows, jnp.full((_NB,), _N_CELLS, jnp.int32)])
    cols_p = jnp.concatenate([x_cols, jnp.zeros((_NB,), jnp.int32)])
    lvals = _log1p_tc(x_vals)
    vals_p = jnp.concatenate([lvals, jnp.zeros((_NB,), jnp.float32)])

    a_flat = _densify(rows_p, cols_p, vals_p, starts, ends)
    a = a_flat.reshape(_N_CELLS, _KP)
    emb_pad = jnp.pad(emb, ((0, _KP - _N_GENES), (0, 0)))
    return _matmul(a, emb_pad)
